# Initial kernel scaffold; baseline (speedup 1.0000x reference)
import jax
import jax.numpy as jnp
from jax.experimental import pallas as pl


def kernel(origin_x, origin_edge_index, origin_edge_attr, origin_batch, frag_x, frag_edge_index, frag_edge_attr, frag_batch, junction_x, junction_edge_index, junction_edge_attr, junction_batch, params):
    raise NotImplementedError("write your pallas kernel here")


# speedup vs baseline: 1.0224x; 1.0224x over previous
import functools

import jax
import jax.numpy as jnp
from jax.experimental import pallas as pl
from jax.experimental.pallas import tpu as pltpu

EPS = 1e-5
B = 256
NEG_SLOPE = 0.01


def _leaky(x, s=0.01):
    return jnp.where(x >= 0, x, s * x)


def _elu(x):
    return jnp.where(x > 0, x, jnp.expm1(x))


def _bn(p, x):
    return p["g"] * x / jnp.sqrt(1.0 + EPS) + p["b"]


def _lin(p, x):
    return x @ p["W"] + p["b"]


def _seg_softmax_noshift(alpha, idx, n):
    e = jnp.exp(alpha)
    s = jax.ops.segment_sum(e, idx, num_segments=n)
    return e / (s[idx] + 1e-16)


def _gru(p, h, x):
    gi = h @ p["Wih"].T + p["bih"]
    gh = x @ p["Whh"].T + p["bhh"]
    ir, iz, inn = jnp.split(gi, 3, axis=-1)
    hr, hz, hn = jnp.split(gh, 3, axis=-1)
    r = jax.nn.sigmoid(ir + hr)
    z = jax.nn.sigmoid(iz + hz)
    nn_ = jnp.tanh(inn + r * hn)
    return (1.0 - z) * nn_ + z * x


def _gate_conv(p, x, src, dst, ea):
    n = x.shape[0]
    H = x.shape[1]
    W1a = p["W1"][:H]
    W1b = p["W1"][H:]
    xw1 = x @ W1a
    eaw = ea @ W1b
    xr = x @ p["att_r"]
    xj = _leaky(xw1[src] + eaw)
    alpha = _leaky(xj @ p["att_l"] + xr[dst])
    a = _seg_softmax_noshift(alpha, dst, n)
    agg = jax.ops.segment_sum(xj * a[:, None], dst, num_segments=n)
    return agg @ p["W2"] + p["bias"]


def _gat_conv(p, x, src, dst):
    n = x.shape[0]
    xp = x @ p["W"]
    a_s = xp @ p["att_src"]
    a_d = xp @ p["att_dst"]
    alpha = _leaky(a_s[src] + a_d[dst])
    a = _seg_softmax_noshift(alpha, dst, n)
    agg = jax.ops.segment_sum(xp[src] * a[:, None], dst, num_segments=n)
    return agg + p["bias"]


def _afp_pool(p, x, src, dst, ea, batch, counts):
    x = _leaky(_lin(p["lin1"], x))
    h = _elu(_gate_conv(p["gate"], x, src, dst, ea))
    x = jax.nn.relu(_gru(p["gru0"], h, x))
    for lp in p["atom"]:
        h = _elu(_gat_conv(lp["conv"], x, src, dst))
        x = jax.nn.relu(_gru(lp["gru"], h, x))
    pooled = jax.ops.segment_sum(x, batch, num_segments=B)
    return pooled @ p["lin2"]["W"] + counts[:, None] * p["lin2"]["b"]


def _channel(p, x, ei, ea, batch, counts):
    src, dst = ei[0], ei[1]
    x = _leaky(_bn(p["node_bn"], _lin(p["node_lin"], x)))
    ea = _leaky(_bn(p["edge_bn"], _lin(p["edge_lin"], ea)))
    heads = [_afp_pool(hp, x, src, dst, ea, batch, counts) for hp in p["heads"]]
    return jax.nn.relu(_bn(p["attend_bn"], _lin(p["attend"], jnp.concatenate(heads, axis=-1))))


def _junction(p, x, ei, ea, batch, counts):
    src, dst = ei[0], ei[1]
    x = _leaky(_bn(p["frag_bn"], _lin(p["frag_lin"], x)))
    heads = []
    for hp in p["heads"]:
        heads.append(_afp_pool(hp["afp"], _lin(hp["proj"], x), src, dst, ea, batch, counts))
    return jax.nn.relu(jnp.mean(jnp.stack(heads, axis=1), axis=1))


def _mlp_kernel(cat_ref, w1_ref, b1_ref, g_ref, bb_ref, w2_ref, b2_ref,
                w3_ref, b3_ref, out_ref):
    cat = cat_ref[...]
    d = _leaky(cat @ w1_ref[...] + b1_ref[...], 1e-07)
    d = g_ref[...] * d / jnp.sqrt(1.0 + EPS) + bb_ref[...]
    h = _leaky(d @ w2_ref[...] + b2_ref[...], 1e-07)
    o = _leaky(h @ w3_ref[...] + b3_ref[...], 1e-07)
    out_ref[...] = o


def _pred_mlp(params, cat):
    p1 = params["pred1"]
    p2 = params["pred2"]
    out = pl.pallas_call(
        _mlp_kernel,
        out_shape=jax.ShapeDtypeStruct((cat.shape[0], 1), jnp.float32),
    )(cat, p1["lin"]["W"], p1["lin"]["b"], p1["bn"]["g"], p1["bn"]["b"],
      p2[0]["W"], p2[0]["b"], p2[1]["W"], p2[1]["b"])
    return out


def kernel(origin_x, origin_edge_index, origin_edge_attr, origin_batch,
           frag_x, frag_edge_index, frag_edge_attr, frag_batch,
           junction_x, junction_edge_index, junction_edge_attr, junction_batch,
           params):
    ones = jnp.ones((origin_batch.shape[0],), jnp.float32)
    c_o = jax.ops.segment_sum(ones, origin_batch, num_segments=B)
    c_f = jax.ops.segment_sum(ones, frag_batch, num_segments=B)
    c_j = jax.ops.segment_sum(ones, junction_batch, num_segments=B)
    g_o = _channel(params["origin"], origin_x, origin_edge_index, origin_edge_attr, origin_batch, c_o)
    g_f = _channel(params["frag"], frag_x, frag_edge_index, frag_edge_attr, frag_batch, c_f)
    g_j = _junction(params["junction"], junction_x, junction_edge_index, junction_edge_attr, junction_batch, c_j)
    cat = jnp.concatenate([g_o, g_f, g_j], axis=-1)
    return _pred_mlp(params, cat)


# speedup vs baseline: 6.3516x; 6.2125x over previous
import functools

import jax
import jax.numpy as jnp
from jax import lax
from jax.experimental import pallas as pl
from jax.experimental.pallas import tpu as pltpu
from jax.experimental.pallas import tpu_sc as plsc

EPS = 1e-5
B = 256
NP = 10112
NREAL = 10000
E = 160000
NC, NS, L = 2, 16, 16
NW = NC * NS
CH = 16
EPAD = ((E + NW * CH - 1) // (NW * CH)) * (NW * CH)
EPER = EPAD // NW
NCHK = EPER // CH
RPS = NP // NS
RN = 632
GN = NP // RN
RE = 512
GE = EPAD // RE

_SC_PARAMS = pltpu.CompilerParams(needs_layout_passes=False)
_MESH = plsc.VectorSubcoreMesh(core_axis_name="c", subcore_axis_name="s",
                               num_cores=NC, num_subcores=NS)


def _leaky(x, s=0.01):
    return jnp.where(x >= 0, x, s * x)


def _elu(x):
    return jnp.where(x > 0, x, jnp.exp(x) - 1.0)


def _lane_bcast(v, i):
    idx = jnp.full((L,), i, jnp.int32)
    dn = lax.GatherDimensionNumbers(offset_dims=(), collapsed_slice_dims=(0,),
                                    start_index_map=(0,))
    return lax.gather(v, idx[:, None], dn, (1,),
                      mode=lax.GatherScatterMode.PROMISE_IN_BOUNDS)


@functools.partial(
    pl.kernel,
    out_type=(jax.ShapeDtypeStruct((NC * NP, 128), jnp.float32),
              jax.ShapeDtypeStruct((NW, NP), jnp.float32)),
    mesh=_MESH, compiler_params=_SC_PARAMS,
    scratch_types=[
        pltpu.VMEM((EPER,), jnp.int32),
        pltpu.VMEM((EPER,), jnp.int32),
        pltpu.VMEM((NP,), jnp.float32),
        pltpu.VMEM((NP,), jnp.float32),
        pltpu.VMEM((NP,), jnp.float32),
        pltpu.VMEM((CH,), jnp.int32),
        pltpu.VMEM((CH,), jnp.int32),
        pltpu.VMEM((CH, 128), jnp.float32),
        pltpu.VMEM((CH, 128), jnp.float32),
        pltpu.VMEM_SHARED((NP, 128), jnp.float32),
        pltpu.SemaphoreType.DMA,
    ],
)
def _gat_sc(xp_hbm, as_hbm, ad_hbm, src_hbm, dst_hbm, zd_hbm, zn_hbm,
            agg_hbm, sp_hbm,
            src_v, dst_v, as_v, ad_v, s_v, sidx_v, didx_v, rows_v, obuf_v,
            agg_sh, sem):
    c = lax.axis_index("c")
    s = lax.axis_index("s")
    wid = s * NC + c
    ebase = wid * EPER
    pltpu.sync_copy(src_hbm.at[pl.ds(ebase, EPER)], src_v)
    pltpu.sync_copy(dst_hbm.at[pl.ds(ebase, EPER)], dst_v)
    pltpu.sync_copy(as_hbm, as_v)
    pltpu.sync_copy(ad_hbm, ad_v)
    pltpu.sync_copy(zn_hbm, s_v)
    r0 = s * RPS
    pltpu.sync_copy(zd_hbm.at[pl.ds(r0, RPS)], agg_sh.at[pl.ds(r0, RPS)])
    plsc.subcore_barrier()

    lane = lax.iota(jnp.int32, CH)

    def body(ci, carry):
        off = ci * CH
        src16 = src_v[pl.ds(off, CH)]
        dst16 = dst_v[pl.ds(off, CH)]
        sidx_v[...] = src16
        didx_v[...] = dst16
        asg = plsc.load_gather(as_v, [src16])
        adg = plsc.load_gather(ad_v, [dst16])
        t = asg + adg
        alpha = jnp.where(t >= 0.0, t, 0.01 * t)
        gi = ebase + off + lane
        e = jnp.where(gi < E, jnp.exp(alpha), 0.0)
        pltpu.async_copy(xp_hbm.at[sidx_v], rows_v, sem).wait()
        for i in range(CH):
            ei = _lane_bcast(e, i)
            for j in range(128 // L):
                obuf_v[i, pl.ds(j * L, L)] = rows_v[i, pl.ds(j * L, L)] * ei
        plsc.addupdate_scatter(s_v, [dst16], e)
        pltpu.sync_copy(obuf_v, agg_sh.at[didx_v], add=True)
        return carry

    lax.fori_loop(0, NCHK, body, 0)
    plsc.subcore_barrier()
    pltpu.sync_copy(agg_sh.at[pl.ds(r0, RPS)],
                    agg_hbm.at[pl.ds(c * NP + r0, RPS)])
    pltpu.sync_copy(s_v, sp_hbm.at[wid])


@functools.partial(
    pl.kernel,
    out_type=(jax.ShapeDtypeStruct((NC * NP, 128), jnp.float32),
              jax.ShapeDtypeStruct((NW, NP), jnp.float32)),
    mesh=_MESH, compiler_params=_SC_PARAMS,
    scratch_types=[
        pltpu.VMEM((EPER,), jnp.int32),
        pltpu.VMEM((EPER,), jnp.int32),
        pltpu.VMEM((NP,), jnp.float32),
        pltpu.VMEM((NP,), jnp.float32),
        pltpu.VMEM((128,), jnp.float32),
        pltpu.VMEM((CH,), jnp.int32),
        pltpu.VMEM((CH,), jnp.int32),
        pltpu.VMEM((CH, 128), jnp.float32),
        pltpu.VMEM((CH, 128), jnp.float32),
        pltpu.VMEM((CH, 128), jnp.float32),
        pltpu.VMEM_SHARED((NP, 128), jnp.float32),
        pltpu.SemaphoreType.DMA,
    ],
)
def _gate_sc(xw1_hbm, xr_hbm, eaw_hbm, src_hbm, dst_hbm, attl_hbm,
             zd_hbm, zn_hbm, agg_hbm, sp_hbm,
             src_v, dst_v, xr_v, s_v, attl_v, sidx_v, didx_v,
             rows_v, ebuf_v, obuf_v, agg_sh, sem):
    c = lax.axis_index("c")
    s = lax.axis_index("s")
    wid = s * NC + c
    ebase = wid * EPER
    pltpu.sync_copy(src_hbm.at[pl.ds(ebase, EPER)], src_v)
    pltpu.sync_copy(dst_hbm.at[pl.ds(ebase, EPER)], dst_v)
    pltpu.sync_copy(xr_hbm, xr_v)
    pltpu.sync_copy(zn_hbm, s_v)
    pltpu.sync_copy(attl_hbm, attl_v)
    r0 = s * RPS
    pltpu.sync_copy(zd_hbm.at[pl.ds(r0, RPS)], agg_sh.at[pl.ds(r0, RPS)])
    plsc.subcore_barrier()

    lane = lax.iota(jnp.int32, CH)
    attl = [attl_v[pl.ds(j * L, L)] for j in range(128 // L)]
    lane_eq = [lane == i for i in range(CH)]

    def body(ci, carry):
        off = ci * CH
        src16 = src_v[pl.ds(off, CH)]
        dst16 = dst_v[pl.ds(off, CH)]
        sidx_v[...] = src16
        didx_v[...] = dst16
        xrg = plsc.load_gather(xr_v, [dst16])
        gi = ebase + off + lane
        valid = (gi < E).astype(jnp.float32)
        pltpu.async_copy(xw1_hbm.at[sidx_v], rows_v, sem).wait()
        pltpu.sync_copy(eaw_hbm.at[pl.ds(ebase + off, CH)], ebuf_v)
        e_vec = jnp.zeros((CH,), jnp.float32)
        for i in range(CH):
            xjs = []
            acc = None
            for j in range(128 // L):
                u = rows_v[i, pl.ds(j * L, L)] + ebuf_v[i, pl.ds(j * L, L)]
                xj = jnp.where(u >= 0.0, u, 0.01 * u)
                xjs.append(xj)
                p = xj * attl[j]
                acc = p if acc is None else acc + p
            d = jnp.sum(acc, axis=0)
            tb = jnp.full((CH,), d, jnp.float32) + _lane_bcast(xrg, i)
            ab = jnp.where(tb >= 0.0, tb, 0.01 * tb)
            eb = jnp.exp(ab) * _lane_bcast(valid, i)
            for j in range(128 // L):
                obuf_v[i, pl.ds(j * L, L)] = xjs[j] * eb
            e_vec = jnp.where(lane_eq[i], eb, e_vec)
        plsc.addupdate_scatter(s_v, [dst16], e_vec)
        pltpu.sync_copy(obuf_v, agg_sh.at[didx_v], add=True)
        return carry

    lax.fori_loop(0, NCHK, body, 0)
    plsc.subcore_barrier()
    pltpu.sync_copy(agg_sh.at[pl.ds(r0, RPS)],
                    agg_hbm.at[pl.ds(c * NP + r0, RPS)])
    pltpu.sync_copy(s_v, sp_hbm.at[wid])


def _rowspec(r, cols):
    return pl.BlockSpec((r, cols), lambda i: (i, 0))


def _wspec(shape):
    return pl.BlockSpec(shape, lambda i: tuple(0 for _ in shape))


def _front_body(has_proj, x_ref, wn_ref, bn_ref, *refs):
    n_in = 8 + (4 if has_proj else 0)
    ins = refs[:n_in]
    outs = refs[n_in:]
    x0 = _leaky(x_ref[...] @ wn_ref[...] + bn_ref[...])
    o = 0
    for h in range(2):
        if has_proj:
            wl1, b1, w1a, attr, wp, bp = ins[h * 6:(h + 1) * 6]
            t = x0 @ wp[...] + bp[...]
        else:
            wl1, b1, w1a, attr = ins[h * 4:(h + 1) * 4]
            t = x0
        xh = _leaky(t @ wl1[...] + b1[...])
        xw1 = xh @ w1a[...]
        xr = jnp.sum(xh * attr[...], axis=1, keepdims=True)
        outs[o][...] = xh
        outs[o + 1][...] = xw1
        outs[o + 2][...] = xr
        o += 3


def _front(x, wn, bn, heads, projs=None):
    has_proj = projs is not None
    ins = [x, wn, bn]
    for h in range(2):
        ins += list(heads[h])
        if has_proj:
            ins += list(projs[h])
    in_specs = [_rowspec(RN, 128), _wspec((128, 128)), _wspec((1, 128))]
    for h in range(2):
        in_specs += [_wspec((128, 128)), _wspec((1, 128)),
                     _wspec((128, 128)), _wspec((1, 128))]
        if has_proj:
            in_specs += [_wspec((128, 128)), _wspec((1, 128))]
    out_shape = []
    out_specs = []
    for h in range(2):
        out_shape += [jax.ShapeDtypeStruct((NP, 128), jnp.float32),
                      jax.ShapeDtypeStruct((NP, 128), jnp.float32),
                      jax.ShapeDtypeStruct((NP, 1), jnp.float32)]
        out_specs += [_rowspec(RN, 128), _rowspec(RN, 128), _rowspec(RN, 1)]
    return pl.pallas_call(
        functools.partial(_front_body, has_proj),
        grid=(GN,), in_specs=in_specs, out_specs=out_specs,
        out_shape=out_shape)(*ins)


def _edge_body(ea_ref, we_ref, be_ref, w1b0_ref, w1b1_ref, o0_ref, o1_ref):
    el = _leaky(ea_ref[...] @ we_ref[...] + be_ref[...])
    o0_ref[...] = el @ w1b0_ref[...]
    o1_ref[...] = el @ w1b1_ref[...]


def _edge(ea, we, be, w1b0, w1b1):
    return pl.pallas_call(
        _edge_body, grid=(GE,),
        in_specs=[_rowspec(RE, 16), _wspec((16, 16)), _wspec((1, 16)),
                  _wspec((16, 128)), _wspec((16, 128))],
        out_specs=[_rowspec(RE, 128), _rowspec(RE, 128)],
        out_shape=[jax.ShapeDtypeStruct((EPAD, 128), jnp.float32),
                   jax.ShapeDtypeStruct((EPAD, 128), jnp.float32)])(
            ea, we, be, w1b0, w1b1)


def _edgej_body(ea_ref, w1b0_ref, w1b1_ref, o0_ref, o1_ref):
    el = ea_ref[...]
    o0_ref[...] = el @ w1b0_ref[...]
    o1_ref[...] = el @ w1b1_ref[...]


def _edgej(ea, w1b0, w1b1):
    return pl.pallas_call(
        _edgej_body, grid=(GE,),
        in_specs=[_rowspec(RE, 16), _wspec((16, 128)), _wspec((16, 128))],
        out_specs=[_rowspec(RE, 128), _rowspec(RE, 128)],
        out_shape=[jax.ShapeDtypeStruct((EPAD, 128), jnp.float32),
                   jax.ShapeDtypeStruct((EPAD, 128), jnp.float32)])(
            ea, w1b0, w1b1)


def _gru_tc(h, x, wiht, bih, whht, bhh):
    gi = h @ wiht[...] + bih[...]
    gh = x @ whht[...] + bhh[...]
    r = jax.nn.sigmoid(gi[:, :128] + gh[:, :128])
    z = jax.nn.sigmoid(gi[:, 128:256] + gh[:, 128:256])
    nn_ = jnp.tanh(gi[:, 256:384] + r * gh[:, 256:384])
    return (1.0 - z) * nn_ + z * x


def _mid_body(a0_ref, a1_ref, sp_ref, xh_ref, w2_ref, gb_ref,
              wiht_ref, bih_ref, whht_ref, bhh_ref, wg_ref,
              asv_ref, adv_ref,
              x1_ref, xp_ref, as_ref, ad_ref):
    agg = a0_ref[...] + a1_ref[...]
    s = jnp.sum(sp_ref[...], axis=1)[:, None]
    u = agg / (s + 1e-16)
    h = _elu(u @ w2_ref[...] + gb_ref[...])
    x1 = jax.nn.relu(_gru_tc(h, xh_ref[...], wiht_ref, bih_ref,
                             whht_ref, bhh_ref))
    xp = x1 @ wg_ref[...]
    x1_ref[...] = x1
    xp_ref[...] = xp
    as_ref[...] = jnp.sum(xp * asv_ref[...], axis=1, keepdims=True)
    ad_ref[...] = jnp.sum(xp * adv_ref[...], axis=1, keepdims=True)


def _mid(aggcat, sp, xh, w2, gb, wiht, bih, whht, bhh, wg, asv, adv):
    return pl.pallas_call(
        _mid_body, grid=(GN,),
        in_specs=[
            pl.BlockSpec((RN, 128), lambda i: (i, 0)),
            pl.BlockSpec((RN, 128), lambda i: (i + GN, 0)),
            pl.BlockSpec((RN, NW), lambda i: (i, 0)),
            _rowspec(RN, 128),
            _wspec((128, 128)), _wspec((1, 128)),
            _wspec((128, 384)), _wspec((1, 384)),
            _wspec((128, 384)), _wspec((1, 384)),
            _wspec((128, 128)), _wspec((1, 128)), _wspec((1, 128)),
        ],
        out_specs=[_rowspec(RN, 128), _rowspec(RN, 128),
                   _rowspec(RN, 1), _rowspec(RN, 1)],
        out_shape=[jax.ShapeDtypeStruct((NP, 128), jnp.float32),
                   jax.ShapeDtypeStruct((NP, 128), jnp.float32),
                   jax.ShapeDtypeStruct((NP, 1), jnp.float32),
                   jax.ShapeDtypeStruct((NP, 1), jnp.float32)])(
            aggcat, aggcat, sp, xh, w2, gb, wiht, bih, whht, bhh,
            wg, asv, adv)


def _post_body(a0_ref, a1_ref, sp_ref, x1_ref, batch_ref, gb_ref,
               wiht_ref, bih_ref, whht_ref, bhh_ref,
               pooled_ref, cnt_ref):
    i = pl.program_id(0)
    agg = a0_ref[...] + a1_ref[...]
    s = jnp.sum(sp_ref[...], axis=1)[:, None]
    u = agg / (s + 1e-16)
    h = _elu(u + gb_ref[...])
    x2 = jax.nn.relu(_gru_tc(h, x1_ref[...], wiht_ref, bih_ref,
                             whht_ref, bhh_ref))
    oh = (batch_ref[...] == lax.broadcasted_iota(jnp.int32, (1, B), 1)
          ).astype(jnp.float32)
    pooled_part = lax.dot_general(oh, x2, (((0,), (0,)), ((), ())),
                                  preferred_element_type=jnp.float32)
    cnt_part = jnp.sum(oh, axis=0)[:, None]

    @pl.when(i == 0)
    def _():
        pooled_ref[...] = jnp.zeros_like(pooled_ref)
        cnt_ref[...] = jnp.zeros_like(cnt_ref)

    pooled_ref[...] += pooled_part
    cnt_ref[...] += cnt_part


def _post(aggcat, sp, x1, batch, gb, wiht, bih, whht, bhh):
    return pl.pallas_call(
        _post_body, grid=(GN,),
        in_specs=[
            pl.BlockSpec((RN, 128), lambda i: (i, 0)),
            pl.BlockSpec((RN, 128), lambda i: (i + GN, 0)),
            pl.BlockSpec((RN, NW), lambda i: (i, 0)),
            _rowspec(RN, 128),
            pl.BlockSpec((RN, 1), lambda i: (i, 0)),
            _wspec((1, 128)),
            _wspec((128, 384)), _wspec((1, 384)),
            _wspec((128, 384)), _wspec((1, 384)),
        ],
        out_specs=[pl.BlockSpec((B, 128), lambda i: (0, 0)),
                   pl.BlockSpec((B, 1), lambda i: (0, 0))],
        out_shape=[jax.ShapeDtypeStruct((B, 128), jnp.float32),
                   jax.ShapeDtypeStruct((B, 1), jnp.float32)])(
            aggcat, aggcat, sp, x1, batch, gb, wiht, bih, whht, bhh)


def _chanfinal_body(p0_ref, p1_ref, cnt_ref, w0_ref, b0_ref, w1_ref, b1_ref,
                    wat_ref, bat_ref, out_ref):
    cnt = cnt_ref[...]
    ph0 = p0_ref[...] @ w0_ref[...] + cnt * b0_ref[...]
    ph1 = p1_ref[...] @ w1_ref[...] + cnt * b1_ref[...]
    cat = jnp.concatenate([ph0, ph1], axis=1)
    out_ref[...] = jax.nn.relu(cat @ wat_ref[...] + bat_ref[...])


def _chanfinal(p0, p1, cnt, w0, b0, w1, b1, wat, bat):
    return pl.pallas_call(
        _chanfinal_body,
        out_shape=jax.ShapeDtypeStruct((B, 128), jnp.float32))(
            p0, p1, cnt, w0, b0, w1, b1, wat, bat)


def _juncfinal_body(p0_ref, p1_ref, cnt_ref, w0_ref, b0_ref, w1_ref, b1_ref,
                    out_ref):
    cnt = cnt_ref[...]
    ph0 = p0_ref[...] @ w0_ref[...] + cnt * b0_ref[...]
    ph1 = p1_ref[...] @ w1_ref[...] + cnt * b1_ref[...]
    out_ref[...] = jax.nn.relu(0.5 * (ph0 + ph1))


def _juncfinal(p0, p1, cnt, w0, b0, w1, b1):
    return pl.pallas_call(
        _juncfinal_body,
        out_shape=jax.ShapeDtypeStruct((B, 128), jnp.float32))(
            p0, p1, cnt, w0, b0, w1, b1)


def _mlp_body(cat_ref, w1_ref, b1_ref, g_ref, bb_ref, w2_ref, b2_ref,
              w3_ref, b3_ref, out_ref):
    d = _leaky(cat_ref[...] @ w1_ref[...] + b1_ref[...], 1e-07)
    d = d * g_ref[...] + bb_ref[...]
    h = _leaky(d @ w2_ref[...] + b2_ref[...], 1e-07)
    out_ref[...] = _leaky(h @ w3_ref[...] + b3_ref[...], 1e-07)


_BNS = 1.0 / jnp.sqrt(1.0 + EPS)


def _fold_lin_bn(lin, bn):
    g = bn["g"] * _BNS
    return lin["W"] * g[None, :], (lin["b"] * g + bn["b"])[None, :]


def _pad_nodes(x):
    return jnp.pad(x, ((0, NP - x.shape[0]), (0, 0)))


def _prep_edges(ei):
    pad = EPAD - ei.shape[1]
    src = jnp.pad(ei[0], (0, pad))
    dst = jnp.pad(ei[1], (0, pad))
    return src, dst


def _head_afp(hp, xh, xw1, xr, eaw, src, dst, batch2, zd, zn):
    gp = hp["gate"]
    agg, sp = _gate_sc(xw1, xr.reshape(NP), eaw, src, dst, gp["att_l"],
                       zd, zn)
    sp = sp.T
    gr0 = hp["gru0"]
    conv = hp["atom"][0]["conv"]
    x1, xp, a_s, a_d = _mid(
        agg, sp, xh, gp["W2"], gp["bias"][None, :],
        gr0["Wih"].T, gr0["bih"][None, :], gr0["Whh"].T, gr0["bhh"][None, :],
        conv["W"], conv["att_src"][None, :], conv["att_dst"][None, :])
    agg2, sp2 = _gat_sc(xp, a_s.reshape(NP), a_d.reshape(NP), src, dst,
                        zd, zn)
    sp2 = sp2.T
    gr1 = hp["atom"][0]["gru"]
    pooled, cnt = _post(
        agg2, sp2, x1, batch2, conv["bias"][None, :],
        gr1["Wih"].T, gr1["bih"][None, :], gr1["Whh"].T, gr1["bhh"][None, :])
    return pooled, cnt


def _channel(p, x, ei, ea, batch, zd, zn, junction=False):
    src, dst = _prep_edges(ei)
    xpad = _pad_nodes(x)
    batch2 = jnp.pad(batch, (0, NP - batch.shape[0]),
                     constant_values=B).astype(jnp.int32)[:, None]
    if junction:
        wn, bn = _fold_lin_bn(p["frag_lin"], p["frag_bn"])
        heads_p = [p["heads"][h]["afp"] for h in range(2)]
        projs = [(p["heads"][h]["proj"]["W"], p["heads"][h]["proj"]["b"][None, :])
                 for h in range(2)]
    else:
        wn, bn = _fold_lin_bn(p["node_lin"], p["node_bn"])
        heads_p = p["heads"]
        projs = None
    heads_in = [(heads_p[h]["lin1"]["W"], heads_p[h]["lin1"]["b"][None, :],
                 heads_p[h]["gate"]["W1"][:128], heads_p[h]["gate"]["att_r"][None, :])
                for h in range(2)]
    fr = _front(xpad, wn, bn, heads_in, projs)
    eapad = jnp.pad(ea, ((0, EPAD - ea.shape[0]), (0, 0)))
    if junction:
        eaw0, eaw1 = _edgej(eapad,
                            heads_p[0]["gate"]["W1"][128:],
                            heads_p[1]["gate"]["W1"][128:])
    else:
        we, be = _fold_lin_bn(p["edge_lin"], p["edge_bn"])
        eaw0, eaw1 = _edge(eapad, we, be,
                           heads_p[0]["gate"]["W1"][128:],
                           heads_p[1]["gate"]["W1"][128:])
    eaws = (eaw0, eaw1)
    pooled = []
    cnt = None
    for h in range(2):
        xh, xw1, xr = fr[h * 3], fr[h * 3 + 1], fr[h * 3 + 2]
        ph, cnt = _head_afp(heads_p[h], xh, xw1, xr, eaws[h], src, dst,
                            batch2, zd, zn)
        pooled.append(ph)
    if junction:
        return _juncfinal(pooled[0], pooled[1], cnt,
                          heads_p[0]["lin2"]["W"], heads_p[0]["lin2"]["b"][None, :],
                          heads_p[1]["lin2"]["W"], heads_p[1]["lin2"]["b"][None, :])
    wat, bat = _fold_lin_bn(p["attend"], p["attend_bn"])
    return _chanfinal(pooled[0], pooled[1], cnt,
                      heads_p[0]["lin2"]["W"], heads_p[0]["lin2"]["b"][None, :],
                      heads_p[1]["lin2"]["W"], heads_p[1]["lin2"]["b"][None, :],
                      wat, bat)


def kernel(origin_x, origin_edge_index, origin_edge_attr, origin_batch,
           frag_x, frag_edge_index, frag_edge_attr, frag_batch,
           junction_x, junction_edge_index, junction_edge_attr, junction_batch,
           params):
    zd = jnp.zeros((NP, 128), jnp.float32)
    zn = jnp.zeros((NP,), jnp.float32)
    g_o = _channel(params["origin"], origin_x, origin_edge_index,
                   origin_edge_attr, origin_batch, zd, zn)
    g_f = _channel(params["frag"], frag_x, frag_edge_index,
                   frag_edge_attr, frag_batch, zd, zn)
    g_j = _channel(params["junction"], junction_x, junction_edge_index,
                   junction_edge_attr, junction_batch, zd, zn, junction=True)
    cat = jnp.concatenate([g_o, g_f, g_j], axis=-1)
    p1 = params["pred1"]
    g = (p1["bn"]["g"] * _BNS)[None, :]
    bb = p1["bn"]["b"][None, :]
    p2 = params["pred2"]
    out = pl.pallas_call(
        _mlp_body,
        out_shape=jax.ShapeDtypeStruct((B, 1), jnp.float32))(
            cat, p1["lin"]["W"], p1["lin"]["b"][None, :], g, bb,
            p2[0]["W"], p2[0]["b"][None, :], p2[1]["W"], p2[1]["b"][None, :])
    return out


# speedup vs baseline: 12.9046x; 2.0317x over previous
import functools

import jax
import jax.numpy as jnp
from jax import lax
from jax.experimental import pallas as pl
from jax.experimental.pallas import tpu as pltpu
from jax.experimental.pallas import tpu_sc as plsc

EPS = 1e-5
B = 256
NP = 10112
NREAL = 10000
E = 160000
NC, NS, L = 2, 16, 16
NW = NC * NS
CH = 16
EPAD = ((E + NW * CH * 2 - 1) // (NW * CH * 2)) * (NW * CH * 2)
EPER = EPAD // NW
NCHK = EPER // CH
RPS = NP // NS
RN = 632
GN = NP // RN
RE = 512
GE = EPAD // RE

_SC_PARAMS = pltpu.CompilerParams(needs_layout_passes=False)
_MESH = plsc.VectorSubcoreMesh(core_axis_name="c", subcore_axis_name="s",
                               num_cores=NC, num_subcores=NS)


def _leaky(x, s=0.01):
    return jnp.where(x >= 0, x, s * x)


def _elu(x):
    return jnp.where(x > 0, x, jnp.exp(x) - 1.0)


def _lane_bcast(v, i):
    idx = jnp.full((L,), i, jnp.int32)
    dn = lax.GatherDimensionNumbers(offset_dims=(), collapsed_slice_dims=(0,),
                                    start_index_map=(0,))
    return lax.gather(v, idx[:, None], dn, (1,),
                      mode=lax.GatherScatterMode.PROMISE_IN_BOUNDS)


@functools.partial(
    pl.kernel,
    out_type=(jax.ShapeDtypeStruct((NC * NP, 128), jnp.float32),
              jax.ShapeDtypeStruct((NW, NP), jnp.float32)),
    mesh=_MESH, compiler_params=_SC_PARAMS,
    scratch_types=[
        pltpu.VMEM((EPER,), jnp.int32),
        pltpu.VMEM((EPER,), jnp.int32),
        pltpu.VMEM((NP,), jnp.float32),
        pltpu.VMEM((NP,), jnp.float32),
        pltpu.VMEM((NP,), jnp.float32),
        pltpu.VMEM((CH,), jnp.int32),
        pltpu.VMEM((CH,), jnp.int32),
        pltpu.VMEM((CH,), jnp.int32),
        pltpu.VMEM((CH,), jnp.int32),
        pltpu.VMEM((CH, 128), jnp.float32),
        pltpu.VMEM((CH, 128), jnp.float32),
        pltpu.VMEM((CH, 128), jnp.float32),
        pltpu.VMEM((CH, 128), jnp.float32),
        pltpu.VMEM_SHARED((NP, 128), jnp.float32),
        pltpu.SemaphoreType.DMA,
        pltpu.SemaphoreType.DMA,
        pltpu.SemaphoreType.DMA,
        pltpu.SemaphoreType.DMA,
    ],
)
def _gat_sc(xp_hbm, as_hbm, ad_hbm, src_hbm, dst_hbm, zd_hbm, zn_hbm,
            agg_hbm, sp_hbm,
            src_v, dst_v, as_v, ad_v, s_v, sidx0_v, sidx1_v, didx0_v, didx1_v,
            rows0_v, rows1_v, obuf0_v, obuf1_v,
            agg_sh, gsem0, gsem1, ssem0, ssem1):
    c = lax.axis_index("c")
    s = lax.axis_index("s")
    wid = s * NC + c
    ebase = wid * EPER
    pltpu.sync_copy(src_hbm.at[pl.ds(ebase, EPER)], src_v)
    pltpu.sync_copy(dst_hbm.at[pl.ds(ebase, EPER)], dst_v)
    pltpu.sync_copy(as_hbm, as_v)
    pltpu.sync_copy(ad_hbm, ad_v)
    pltpu.sync_copy(zn_hbm, s_v)
    r0 = s * RPS
    pltpu.sync_copy(zd_hbm.at[pl.ds(r0, RPS)], agg_sh.at[pl.ds(r0, RPS)])
    plsc.subcore_barrier()

    lane = lax.iota(jnp.int32, CH)
    sidx = (sidx0_v, sidx1_v)
    didx = (didx0_v, didx1_v)
    rows = (rows0_v, rows1_v)
    obuf = (obuf0_v, obuf1_v)
    gsem = (gsem0, gsem1)
    ssem = (ssem0, ssem1)

    for b in range(2):
        sidx[b][...] = src_v[pl.ds(b * CH, CH)]
        pltpu.async_copy(xp_hbm.at[sidx[b]], rows[b], gsem[b])

    def body(it, carry):
        for b in range(2):
            ci = it * 2 + b

            @pl.when(it > 0)
            def _():
                pltpu.make_async_copy(obuf[b], agg_sh.at[didx[b]],
                                      ssem[b]).wait()

            off = ci * CH
            dst16 = dst_v[pl.ds(off, CH)]
            didx[b][...] = dst16
            src16 = src_v[pl.ds(off, CH)]
            asg = plsc.load_gather(as_v, [src16])
            adg = plsc.load_gather(ad_v, [dst16])
            t = asg + adg
            alpha = jnp.where(t >= 0.0, t, 0.01 * t)
            gi = ebase + off + lane
            e = jnp.where(gi < E, jnp.exp(alpha), 0.0)
            pltpu.make_async_copy(xp_hbm.at[sidx[b]], rows[b], gsem[b]).wait()
            for i in range(CH):
                ei = _lane_bcast(e, i)
                for j in range(128 // L):
                    obuf[b][i, pl.ds(j * L, L)] = \
                        rows[b][i, pl.ds(j * L, L)] * ei
            plsc.addupdate_scatter(s_v, [dst16], e)
            pltpu.async_copy(obuf[b], agg_sh.at[didx[b]], ssem[b], add=True)

            @pl.when(ci + 2 < NCHK)
            def _():
                sidx[b][...] = src_v[pl.ds((ci + 2) * CH, CH)]
                pltpu.async_copy(xp_hbm.at[sidx[b]], rows[b], gsem[b])

        return carry

    lax.fori_loop(0, NCHK // 2, body, 0)
    for b in range(2):
        pltpu.make_async_copy(obuf[b], agg_sh.at[didx[b]], ssem[b]).wait()
    plsc.subcore_barrier()
    pltpu.sync_copy(agg_sh.at[pl.ds(r0, RPS)],
                    agg_hbm.at[pl.ds(c * NP + r0, RPS)])
    pltpu.sync_copy(s_v, sp_hbm.at[wid])


@functools.partial(
    pl.kernel,
    out_type=(jax.ShapeDtypeStruct((NC * NP, 128), jnp.float32),
              jax.ShapeDtypeStruct((NW, NP), jnp.float32)),
    mesh=_MESH, compiler_params=_SC_PARAMS,
    scratch_types=[
        pltpu.VMEM((EPER,), jnp.int32),
        pltpu.VMEM((EPER,), jnp.int32),
        pltpu.VMEM((NP,), jnp.float32),
        pltpu.VMEM((NP,), jnp.float32),
        pltpu.VMEM((128,), jnp.float32),
        pltpu.VMEM((CH,), jnp.int32),
        pltpu.VMEM((CH,), jnp.int32),
        pltpu.VMEM((CH,), jnp.int32),
        pltpu.VMEM((CH,), jnp.int32),
        pltpu.VMEM((CH, 128), jnp.float32),
        pltpu.VMEM((CH, 128), jnp.float32),
        pltpu.VMEM((CH, 128), jnp.float32),
        pltpu.VMEM((CH, 128), jnp.float32),
        pltpu.VMEM((CH, 128), jnp.float32),
        pltpu.VMEM((CH, 128), jnp.float32),
        pltpu.VMEM_SHARED((NP, 128), jnp.float32),
        pltpu.SemaphoreType.DMA,
        pltpu.SemaphoreType.DMA,
        pltpu.SemaphoreType.DMA,
        pltpu.SemaphoreType.DMA,
        pltpu.SemaphoreType.DMA,
        pltpu.SemaphoreType.DMA,
    ],
)
def _gate_sc(xw1_hbm, xr_hbm, eaw_hbm, src_hbm, dst_hbm, attl_hbm,
             zd_hbm, zn_hbm, agg_hbm, sp_hbm,
             src_v, dst_v, xr_v, s_v, attl_v, sidx0_v, sidx1_v,
             didx0_v, didx1_v, rows0_v, rows1_v, ebuf0_v, ebuf1_v,
             obuf0_v, obuf1_v, agg_sh,
             gsem0, gsem1, esem0, esem1, ssem0, ssem1):
    c = lax.axis_index("c")
    s = lax.axis_index("s")
    wid = s * NC + c
    ebase = wid * EPER
    pltpu.sync_copy(src_hbm.at[pl.ds(ebase, EPER)], src_v)
    pltpu.sync_copy(dst_hbm.at[pl.ds(ebase, EPER)], dst_v)
    pltpu.sync_copy(xr_hbm, xr_v)
    pltpu.sync_copy(zn_hbm, s_v)
    pltpu.sync_copy(attl_hbm, attl_v)
    r0 = s * RPS
    pltpu.sync_copy(zd_hbm.at[pl.ds(r0, RPS)], agg_sh.at[pl.ds(r0, RPS)])
    plsc.subcore_barrier()

    lane = lax.iota(jnp.int32, CH)
    attl = [attl_v[pl.ds(j * L, L)] for j in range(128 // L)]
    lane_eq = [lane == i for i in range(CH)]
    sidx = (sidx0_v, sidx1_v)
    didx = (didx0_v, didx1_v)
    rows = (rows0_v, rows1_v)
    ebuf = (ebuf0_v, ebuf1_v)
    obuf = (obuf0_v, obuf1_v)
    gsem = (gsem0, gsem1)
    esem = (esem0, esem1)
    ssem = (ssem0, ssem1)

    for b in range(2):
        sidx[b][...] = src_v[pl.ds(b * CH, CH)]
        pltpu.async_copy(xw1_hbm.at[sidx[b]], rows[b], gsem[b])
        pltpu.async_copy(eaw_hbm.at[pl.ds(ebase + b * CH, CH)],
                         ebuf[b], esem[b])

    def body(it, carry):
        for b in range(2):
            ci = it * 2 + b

            @pl.when(it > 0)
            def _():
                pltpu.make_async_copy(obuf[b], agg_sh.at[didx[b]],
                                      ssem[b]).wait()

            off = ci * CH
            dst16 = dst_v[pl.ds(off, CH)]
            didx[b][...] = dst16
            xrg = plsc.load_gather(xr_v, [dst16])
            gi = ebase + off + lane
            valid = (gi < E).astype(jnp.float32)
            pltpu.make_async_copy(xw1_hbm.at[sidx[b]], rows[b],
                                  gsem[b]).wait()
            pltpu.make_async_copy(eaw_hbm.at[pl.ds(ebase + off, CH)],
                                  ebuf[b], esem[b]).wait()
            e_vec = jnp.zeros((CH,), jnp.float32)
            for i in range(CH):
                xjs = []
                acc = None
                for j in range(128 // L):
                    u = rows[b][i, pl.ds(j * L, L)] + \
                        ebuf[b][i, pl.ds(j * L, L)]
                    xj = jnp.where(u >= 0.0, u, 0.01 * u)
                    xjs.append(xj)
                    p = xj * attl[j]
                    acc = p if acc is None else acc + p
                d = jnp.sum(acc, axis=0)
                tb = jnp.full((CH,), d, jnp.float32) + _lane_bcast(xrg, i)
                ab = jnp.where(tb >= 0.0, tb, 0.01 * tb)
                eb = jnp.exp(ab) * _lane_bcast(valid, i)
                for j in range(128 // L):
                    obuf[b][i, pl.ds(j * L, L)] = xjs[j] * eb
                e_vec = jnp.where(lane_eq[i], eb, e_vec)
            plsc.addupdate_scatter(s_v, [dst16], e_vec)
            pltpu.async_copy(obuf[b], agg_sh.at[didx[b]], ssem[b], add=True)

            @pl.when(ci + 2 < NCHK)
            def _():
                sidx[b][...] = src_v[pl.ds((ci + 2) * CH, CH)]
                pltpu.async_copy(xw1_hbm.at[sidx[b]], rows[b], gsem[b])
                pltpu.async_copy(eaw_hbm.at[pl.ds(ebase + (ci + 2) * CH, CH)],
                                 ebuf[b], esem[b])

        return carry

    lax.fori_loop(0, NCHK // 2, body, 0)
    for b in range(2):
        pltpu.make_async_copy(obuf[b], agg_sh.at[didx[b]], ssem[b]).wait()
    plsc.subcore_barrier()
    pltpu.sync_copy(agg_sh.at[pl.ds(r0, RPS)],
                    agg_hbm.at[pl.ds(c * NP + r0, RPS)])
    pltpu.sync_copy(s_v, sp_hbm.at[wid])


def _rowspec(r, cols):
    return pl.BlockSpec((r, cols), lambda i: (i, 0))


def _wspec(shape):
    return pl.BlockSpec(shape, lambda i: tuple(0 for _ in shape))


def _front_body(has_proj, x_ref, wn_ref, bn_ref, *refs):
    n_in = 8 + (4 if has_proj else 0)
    ins = refs[:n_in]
    outs = refs[n_in:]
    x0 = _leaky(x_ref[...] @ wn_ref[...] + bn_ref[...])
    o = 0
    for h in range(2):
        if has_proj:
            wl1, b1, w1a, attr, wp, bp = ins[h * 6:(h + 1) * 6]
            t = x0 @ wp[...] + bp[...]
        else:
            wl1, b1, w1a, attr = ins[h * 4:(h + 1) * 4]
            t = x0
        xh = _leaky(t @ wl1[...] + b1[...])
        xw1 = xh @ w1a[...]
        xr = jnp.sum(xh * attr[...], axis=1, keepdims=True)
        outs[o][...] = xh
        outs[o + 1][...] = xw1
        outs[o + 2][...] = xr
        o += 3


def _front(x, wn, bn, heads, projs=None):
    has_proj = projs is not None
    ins = [x, wn, bn]
    for h in range(2):
        ins += list(heads[h])
        if has_proj:
            ins += list(projs[h])
    in_specs = [_rowspec(RN, 128), _wspec((128, 128)), _wspec((1, 128))]
    for h in range(2):
        in_specs += [_wspec((128, 128)), _wspec((1, 128)),
                     _wspec((128, 128)), _wspec((1, 128))]
        if has_proj:
            in_specs += [_wspec((128, 128)), _wspec((1, 128))]
    out_shape = []
    out_specs = []
    for h in range(2):
        out_shape += [jax.ShapeDtypeStruct((NP, 128), jnp.float32),
                      jax.ShapeDtypeStruct((NP, 128), jnp.float32),
                      jax.ShapeDtypeStruct((NP, 1), jnp.float32)]
        out_specs += [_rowspec(RN, 128), _rowspec(RN, 128), _rowspec(RN, 1)]
    return pl.pallas_call(
        functools.partial(_front_body, has_proj),
        grid=(GN,), in_specs=in_specs, out_specs=out_specs,
        out_shape=out_shape)(*ins)


def _edge_body(ea_ref, we_ref, be_ref, w1b0_ref, w1b1_ref, o0_ref, o1_ref):
    el = _leaky(ea_ref[...] @ we_ref[...] + be_ref[...])
    o0_ref[...] = el @ w1b0_ref[...]
    o1_ref[...] = el @ w1b1_ref[...]


def _edge(ea, we, be, w1b0, w1b1):
    return pl.pallas_call(
        _edge_body, grid=(GE,),
        in_specs=[_rowspec(RE, 16), _wspec((16, 16)), _wspec((1, 16)),
                  _wspec((16, 128)), _wspec((16, 128))],
        out_specs=[_rowspec(RE, 128), _rowspec(RE, 128)],
        out_shape=[jax.ShapeDtypeStruct((EPAD, 128), jnp.float32),
                   jax.ShapeDtypeStruct((EPAD, 128), jnp.float32)])(
            ea, we, be, w1b0, w1b1)


def _edgej_body(ea_ref, w1b0_ref, w1b1_ref, o0_ref, o1_ref):
    el = ea_ref[...]
    o0_ref[...] = el @ w1b0_ref[...]
    o1_ref[...] = el @ w1b1_ref[...]


def _edgej(ea, w1b0, w1b1):
    return pl.pallas_call(
        _edgej_body, grid=(GE,),
        in_specs=[_rowspec(RE, 16), _wspec((16, 128)), _wspec((16, 128))],
        out_specs=[_rowspec(RE, 128), _rowspec(RE, 128)],
        out_shape=[jax.ShapeDtypeStruct((EPAD, 128), jnp.float32),
                   jax.ShapeDtypeStruct((EPAD, 128), jnp.float32)])(
            ea, w1b0, w1b1)


def _gru_tc(h, x, wiht, bih, whht, bhh):
    gi = h @ wiht[...] + bih[...]
    gh = x @ whht[...] + bhh[...]
    r = jax.nn.sigmoid(gi[:, :128] + gh[:, :128])
    z = jax.nn.sigmoid(gi[:, 128:256] + gh[:, 128:256])
    nn_ = jnp.tanh(gi[:, 256:384] + r * gh[:, 256:384])
    return (1.0 - z) * nn_ + z * x


def _mid_body(a0_ref, a1_ref, sp_ref, xh_ref, w2_ref, gb_ref,
              wiht_ref, bih_ref, whht_ref, bhh_ref, wg_ref,
              asv_ref, adv_ref,
              x1_ref, xp_ref, as_ref, ad_ref):
    agg = a0_ref[...] + a1_ref[...]
    s = jnp.sum(sp_ref[...], axis=1)[:, None]
    u = agg / (s + 1e-16)
    h = _elu(u @ w2_ref[...] + gb_ref[...])
    x1 = jax.nn.relu(_gru_tc(h, xh_ref[...], wiht_ref, bih_ref,
                             whht_ref, bhh_ref))
    xp = x1 @ wg_ref[...]
    x1_ref[...] = x1
    xp_ref[...] = xp
    as_ref[...] = jnp.sum(xp * asv_ref[...], axis=1, keepdims=True)
    ad_ref[...] = jnp.sum(xp * adv_ref[...], axis=1, keepdims=True)


def _mid(aggcat, sp, xh, w2, gb, wiht, bih, whht, bhh, wg, asv, adv):
    return pl.pallas_call(
        _mid_body, grid=(GN,),
        in_specs=[
            pl.BlockSpec((RN, 128), lambda i: (i, 0)),
            pl.BlockSpec((RN, 128), lambda i: (i + GN, 0)),
            pl.BlockSpec((RN, NW), lambda i: (i, 0)),
            _rowspec(RN, 128),
            _wspec((128, 128)), _wspec((1, 128)),
            _wspec((128, 384)), _wspec((1, 384)),
            _wspec((128, 384)), _wspec((1, 384)),
            _wspec((128, 128)), _wspec((1, 128)), _wspec((1, 128)),
        ],
        out_specs=[_rowspec(RN, 128), _rowspec(RN, 128),
                   _rowspec(RN, 1), _rowspec(RN, 1)],
        out_shape=[jax.ShapeDtypeStruct((NP, 128), jnp.float32),
                   jax.ShapeDtypeStruct((NP, 128), jnp.float32),
                   jax.ShapeDtypeStruct((NP, 1), jnp.float32),
                   jax.ShapeDtypeStruct((NP, 1), jnp.float32)])(
            aggcat, aggcat, sp, xh, w2, gb, wiht, bih, whht, bhh,
            wg, asv, adv)


def _post_body(a0_ref, a1_ref, sp_ref, x1_ref, batch_ref, gb_ref,
               wiht_ref, bih_ref, whht_ref, bhh_ref,
               pooled_ref, cnt_ref):
    i = pl.program_id(0)
    agg = a0_ref[...] + a1_ref[...]
    s = jnp.sum(sp_ref[...], axis=1)[:, None]
    u = agg / (s + 1e-16)
    h = _elu(u + gb_ref[...])
    x2 = jax.nn.relu(_gru_tc(h, x1_ref[...], wiht_ref, bih_ref,
                             whht_ref, bhh_ref))
    oh = (batch_ref[...] == lax.broadcasted_iota(jnp.int32, (1, B), 1)
          ).astype(jnp.float32)
    pooled_part = lax.dot_general(oh, x2, (((0,), (0,)), ((), ())),
                                  preferred_element_type=jnp.float32)
    cnt_part = jnp.sum(oh, axis=0)[:, None]

    @pl.when(i == 0)
    def _():
        pooled_ref[...] = jnp.zeros_like(pooled_ref)
        cnt_ref[...] = jnp.zeros_like(cnt_ref)

    pooled_ref[...] += pooled_part
    cnt_ref[...] += cnt_part


def _post(aggcat, sp, x1, batch, gb, wiht, bih, whht, bhh):
    return pl.pallas_call(
        _post_body, grid=(GN,),
        in_specs=[
            pl.BlockSpec((RN, 128), lambda i: (i, 0)),
            pl.BlockSpec((RN, 128), lambda i: (i + GN, 0)),
            pl.BlockSpec((RN, NW), lambda i: (i, 0)),
            _rowspec(RN, 128),
            pl.BlockSpec((RN, 1), lambda i: (i, 0)),
            _wspec((1, 128)),
            _wspec((128, 384)), _wspec((1, 384)),
            _wspec((128, 384)), _wspec((1, 384)),
        ],
        out_specs=[pl.BlockSpec((B, 128), lambda i: (0, 0)),
                   pl.BlockSpec((B, 1), lambda i: (0, 0))],
        out_shape=[jax.ShapeDtypeStruct((B, 128), jnp.float32),
                   jax.ShapeDtypeStruct((B, 1), jnp.float32)])(
            aggcat, aggcat, sp, x1, batch, gb, wiht, bih, whht, bhh)


def _chanfinal_body(p0_ref, p1_ref, cnt_ref, w0_ref, b0_ref, w1_ref, b1_ref,
                    wat_ref, bat_ref, out_ref):
    cnt = cnt_ref[...]
    ph0 = p0_ref[...] @ w0_ref[...] + cnt * b0_ref[...]
    ph1 = p1_ref[...] @ w1_ref[...] + cnt * b1_ref[...]
    cat = jnp.concatenate([ph0, ph1], axis=1)
    out_ref[...] = jax.nn.relu(cat @ wat_ref[...] + bat_ref[...])


def _chanfinal(p0, p1, cnt, w0, b0, w1, b1, wat, bat):
    return pl.pallas_call(
        _chanfinal_body,
        out_shape=jax.ShapeDtypeStruct((B, 128), jnp.float32))(
            p0, p1, cnt, w0, b0, w1, b1, wat, bat)


def _juncfinal_body(p0_ref, p1_ref, cnt_ref, w0_ref, b0_ref, w1_ref, b1_ref,
                    out_ref):
    cnt = cnt_ref[...]
    ph0 = p0_ref[...] @ w0_ref[...] + cnt * b0_ref[...]
    ph1 = p1_ref[...] @ w1_ref[...] + cnt * b1_ref[...]
    out_ref[...] = jax.nn.relu(0.5 * (ph0 + ph1))


def _juncfinal(p0, p1, cnt, w0, b0, w1, b1):
    return pl.pallas_call(
        _juncfinal_body,
        out_shape=jax.ShapeDtypeStruct((B, 128), jnp.float32))(
            p0, p1, cnt, w0, b0, w1, b1)


def _mlp_body(cat_ref, w1_ref, b1_ref, g_ref, bb_ref, w2_ref, b2_ref,
              w3_ref, b3_ref, out_ref):
    d = _leaky(cat_ref[...] @ w1_ref[...] + b1_ref[...], 1e-07)
    d = d * g_ref[...] + bb_ref[...]
    h = _leaky(d @ w2_ref[...] + b2_ref[...], 1e-07)
    out_ref[...] = _leaky(h @ w3_ref[...] + b3_ref[...], 1e-07)


_BNS = 1.0 / jnp.sqrt(1.0 + EPS)


def _fold_lin_bn(lin, bn):
    g = bn["g"] * _BNS
    return lin["W"] * g[None, :], (lin["b"] * g + bn["b"])[None, :]


def _pad_nodes(x):
    return jnp.pad(x, ((0, NP - x.shape[0]), (0, 0)))


def _prep_edges(ei):
    pad = EPAD - ei.shape[1]
    src = jnp.pad(ei[0], (0, pad))
    dst = jnp.pad(ei[1], (0, pad))
    return src, dst


def _head_afp(hp, xh, xw1, xr, eaw, src, dst, batch2, zd, zn):
    gp = hp["gate"]
    agg, sp = _gate_sc(xw1, xr.reshape(NP), eaw, src, dst, gp["att_l"],
                       zd, zn)
    sp = sp.T
    gr0 = hp["gru0"]
    conv = hp["atom"][0]["conv"]
    x1, xp, a_s, a_d = _mid(
        agg, sp, xh, gp["W2"], gp["bias"][None, :],
        gr0["Wih"].T, gr0["bih"][None, :], gr0["Whh"].T, gr0["bhh"][None, :],
        conv["W"], conv["att_src"][None, :], conv["att_dst"][None, :])
    agg2, sp2 = _gat_sc(xp, a_s.reshape(NP), a_d.reshape(NP), src, dst,
                        zd, zn)
    sp2 = sp2.T
    gr1 = hp["atom"][0]["gru"]
    pooled, cnt = _post(
        agg2, sp2, x1, batch2, conv["bias"][None, :],
        gr1["Wih"].T, gr1["bih"][None, :], gr1["Whh"].T, gr1["bhh"][None, :])
    return pooled, cnt


def _channel(p, x, ei, ea, batch, zd, zn, junction=False):
    src, dst = _prep_edges(ei)
    xpad = _pad_nodes(x)
    batch2 = jnp.pad(batch, (0, NP - batch.shape[0]),
                     constant_values=B).astype(jnp.int32)[:, None]
    if junction:
        wn, bn = _fold_lin_bn(p["frag_lin"], p["frag_bn"])
        heads_p = [p["heads"][h]["afp"] for h in range(2)]
        projs = [(p["heads"][h]["proj"]["W"], p["heads"][h]["proj"]["b"][None, :])
                 for h in range(2)]
    else:
        wn, bn = _fold_lin_bn(p["node_lin"], p["node_bn"])
        heads_p = p["heads"]
        projs = None
    heads_in = [(heads_p[h]["lin1"]["W"], heads_p[h]["lin1"]["b"][None, :],
                 heads_p[h]["gate"]["W1"][:128], heads_p[h]["gate"]["att_r"][None, :])
                for h in range(2)]
    fr = _front(xpad, wn, bn, heads_in, projs)
    eapad = jnp.pad(ea, ((0, EPAD - ea.shape[0]), (0, 0)))
    if junction:
        eaw0, eaw1 = _edgej(eapad,
                            heads_p[0]["gate"]["W1"][128:],
                            heads_p[1]["gate"]["W1"][128:])
    else:
        we, be = _fold_lin_bn(p["edge_lin"], p["edge_bn"])
        eaw0, eaw1 = _edge(eapad, we, be,
                           heads_p[0]["gate"]["W1"][128:],
                           heads_p[1]["gate"]["W1"][128:])
    eaws = (eaw0, eaw1)
    pooled = []
    cnt = None
    for h in range(2):
        xh, xw1, xr = fr[h * 3], fr[h * 3 + 1], fr[h * 3 + 2]
        ph, cnt = _head_afp(heads_p[h], xh, xw1, xr, eaws[h], src, dst,
                            batch2, zd, zn)
        pooled.append(ph)
    if junction:
        return _juncfinal(pooled[0], pooled[1], cnt,
                          heads_p[0]["lin2"]["W"], heads_p[0]["lin2"]["b"][None, :],
                          heads_p[1]["lin2"]["W"], heads_p[1]["lin2"]["b"][None, :])
    wat, bat = _fold_lin_bn(p["attend"], p["attend_bn"])
    return _chanfinal(pooled[0], pooled[1], cnt,
                      heads_p[0]["lin2"]["W"], heads_p[0]["lin2"]["b"][None, :],
                      heads_p[1]["lin2"]["W"], heads_p[1]["lin2"]["b"][None, :],
                      wat, bat)


def kernel(origin_x, origin_edge_index, origin_edge_attr, origin_batch,
           frag_x, frag_edge_index, frag_edge_attr, frag_batch,
           junction_x, junction_edge_index, junction_edge_attr, junction_batch,
           params):
    zd = jnp.zeros((NP, 128), jnp.float32)
    zn = jnp.zeros((NP,), jnp.float32)
    g_o = _channel(params["origin"], origin_x, origin_edge_index,
                   origin_edge_attr, origin_batch, zd, zn)
    g_f = _channel(params["frag"], frag_x, frag_edge_index,
                   frag_edge_attr, frag_batch, zd, zn)
    g_j = _channel(params["junction"], junction_x, junction_edge_index,
                   junction_edge_attr, junction_batch, zd, zn, junction=True)
    cat = jnp.concatenate([g_o, g_f, g_j], axis=-1)
    p1 = params["pred1"]
    g = (p1["bn"]["g"] * _BNS)[None, :]
    bb = p1["bn"]["b"][None, :]
    p2 = params["pred2"]
    out = pl.pallas_call(
        _mlp_body,
        out_shape=jax.ShapeDtypeStruct((B, 1), jnp.float32))(
            cat, p1["lin"]["W"], p1["lin"]["b"][None, :], g, bb,
            p2[0]["W"], p2[0]["b"][None, :], p2[1]["W"], p2[1]["b"][None, :])
    return out
